# Initial kernel scaffold; baseline (speedup 1.0000x reference)
import functools

import jax
import jax.numpy as jnp
from jax import lax
from jax.experimental import pallas as pl
from jax.experimental.pallas import tpu as pltpu
from jax.experimental.pallas import tpu_sc as plsc

D = 32
B = 16384 * 50
NC, NS = 2, 16
NW = NC * NS
B_PER_W = B // NW
CHUNK = 1280
N_CHUNKS = B_PER_W // CHUNK
NBUF = 2


def _gather_kernel(idx_hbm, table_hbm, out_hbm, idx_v, rows_v, gsem0, gsem1):
    wid = lax.axis_index("s") * NC + lax.axis_index("c")
    base = wid * B_PER_W
    pltpu.sync_copy(idx_hbm.at[wid], idx_v)
    sems = (gsem0, gsem1)
    for b in range(NBUF):
        pltpu.async_copy(table_hbm.at[idx_v.at[b]], rows_v.at[b], sems[b])

    @pl.loop(0, N_CHUNKS, step=NBUF)
    def _body(g):
        for b in range(NBUF):
            c = g + b
            pltpu.make_async_copy(
                table_hbm.at[idx_v.at[c]], rows_v.at[b], sems[b]
            ).wait()
            pltpu.sync_copy(
                rows_v.at[b], out_hbm.at[pl.ds(base + c * CHUNK, CHUNK)]
            )

            @pl.when(c + NBUF < N_CHUNKS)
            def _():
                pltpu.async_copy(
                    table_hbm.at[idx_v.at[c + NBUF]], rows_v.at[b], sems[b]
                )


@jax.jit
def _lookup(idx3, weight):
    mesh = plsc.VectorSubcoreMesh(core_axis_name="c", subcore_axis_name="s")
    run = functools.partial(
        pl.kernel,
        mesh=mesh,
        out_type=jax.ShapeDtypeStruct((B, D), jnp.float32),
        scratch_types=[
            pltpu.VMEM((N_CHUNKS, CHUNK), jnp.int32),
            pltpu.VMEM((NBUF, CHUNK, D), jnp.float32),
            pltpu.SemaphoreType.DMA,
            pltpu.SemaphoreType.DMA,
        ],
    )(_gather_kernel)
    return run(idx3, weight)


def kernel(x, weight):
    idx3 = x.reshape(NW, N_CHUNKS, CHUNK).astype(jnp.int32)
    out = _lookup(idx3, weight)
    return out.reshape(x.shape + (D,))


# speedup vs baseline: 1.1138x; 1.1138x over previous
import functools

import jax
import jax.numpy as jnp
from jax import lax
from jax.experimental import pallas as pl
from jax.experimental.pallas import tpu as pltpu
from jax.experimental.pallas import tpu_sc as plsc

D = 32
B = 16384 * 50
NC, NS = 2, 16
NW = NC * NS
B_PER_W = B // NW
CHUNK = 1280
N_CHUNKS = B_PER_W // CHUNK
NBUF = 2


def _gather_kernel(idx_hbm, table_hbm, out_hbm, idx_v, rows_v, gsem0, gsem1):
    wid = lax.axis_index("s") * NC + lax.axis_index("c")
    base = wid * B_PER_W
    pltpu.sync_copy(idx_hbm.at[wid], idx_v)
    sems = (gsem0, gsem1)
    for b in range(NBUF):
        pltpu.async_copy(table_hbm.at[idx_v.at[b]], rows_v.at[b], sems[b])

    @pl.loop(0, N_CHUNKS, step=NBUF)
    def _body(g):
        for b in range(NBUF):
            c = g + b
            pltpu.make_async_copy(
                table_hbm.at[idx_v.at[c]], rows_v.at[b], sems[b]
            ).wait()
            pltpu.sync_copy(
                rows_v.at[b], out_hbm.at[pl.ds(base + c * CHUNK, CHUNK)]
            )

            @pl.when(c + NBUF < N_CHUNKS)
            def _():
                pltpu.async_copy(
                    table_hbm.at[idx_v.at[c + NBUF]], rows_v.at[b], sems[b]
                )


@jax.jit
def _lookup(idx3, weight):
    mesh = plsc.VectorSubcoreMesh(core_axis_name="c", subcore_axis_name="s")
    run = functools.partial(
        pl.kernel,
        mesh=mesh,
        out_type=jax.ShapeDtypeStruct((B, D), jnp.float32),
        scratch_types=[
            pltpu.VMEM((N_CHUNKS, CHUNK), jnp.int32),
            pltpu.VMEM((NBUF, CHUNK, D), jnp.float32),
            pltpu.SemaphoreType.DMA,
            pltpu.SemaphoreType.DMA,
        ],
        compiler_params=pltpu.CompilerParams(use_tc_tiling_on_sc=False),
    )(_gather_kernel)
    return run(idx3, weight)


def kernel(x, weight):
    idx3 = x.reshape(NW, N_CHUNKS, CHUNK).astype(jnp.int32)
    out = _lookup(idx3, weight)
    return out.reshape(x.shape + (D,))


# speedup vs baseline: 1.8674x; 1.6766x over previous
import functools

import jax
import jax.numpy as jnp
from jax import lax
from jax.experimental import pallas as pl
from jax.experimental.pallas import tpu as pltpu
from jax.experimental.pallas import tpu_sc as plsc

J = 50
NI = 16384
K = 32
CB = 512
NBLK = NI // CB
NTILES = J * NBLK
NC, NS = 2, 16
NW = NC * NS
TPW = NTILES // NW
NBUF = 2
BAND = 8 * CB


def _emb_kernel(xt_hbm, table_hbm, out5_hbm, idx_all, rows_v, tout_v,
                gsem0, gsem1, wsem0, wsem1):
    wid = lax.axis_index("s") * NC + lax.axis_index("c")
    t0 = wid * TPW
    gsems = (gsem0, gsem1)
    wsems = (wsem0, wsem1)
    pltpu.sync_copy(xt_hbm.at[pl.ds(wid * (TPW * CB), TPW * CB)], idx_all)
    lane = lax.iota(jnp.int32, 16)
    vbase = ((lane >> 3) << 12) + ((lane & 7) << 7)
    for b in range(NBUF):
        pltpu.async_copy(
            table_hbm.at[idx_all.at[pl.ds(b * CB, CB)]], rows_v.at[b], gsems[b]
        )

    @pl.loop(0, TPW, step=NBUF)
    def _body(g):
        for b in range(NBUF):
            n = g + b
            t = t0 + n
            j = t // NBLK
            c = t - j * NBLK
            pltpu.make_async_copy(
                table_hbm.at[idx_all.at[pl.ds(n * CB, CB)]], rows_v.at[b],
                gsems[b],
            ).wait()

            @pl.when(g > 0)
            def _():
                for r in range(4):
                    pltpu.make_async_copy(
                        tout_v.at[b, pl.ds(r * BAND, BAND)],
                        out5_hbm.at[0, r, 0], wsems[b],
                    ).wait()

            for q in range(CB // 128):
                @pl.loop(0, 128, unroll=8, init_carry=vbase + q * 1024)
                def _tr(t, idxv):
                    i = q * 128 + t
                    lo = rows_v[b, i, pl.ds(0, 16)]
                    hi = rows_v[b, i, pl.ds(16, 16)]
                    plsc.store_scatter(tout_v.at[b], [idxv], lo)
                    plsc.store_scatter(tout_v.at[b], [idxv + 2 * BAND], hi)
                    return idxv + 1

            for r in range(4):
                pltpu.async_copy(
                    tout_v.at[b, pl.ds(r * BAND, BAND)],
                    out5_hbm.at[j, r, c], wsems[b],
                )

            @pl.when(n + NBUF < TPW)
            def _():
                pltpu.async_copy(
                    table_hbm.at[idx_all.at[pl.ds((n + NBUF) * CB, CB)]],
                    rows_v.at[b], gsems[b],
                )

    for b in range(NBUF):
        for r in range(4):
            pltpu.make_async_copy(
                tout_v.at[b, pl.ds(r * BAND, BAND)], out5_hbm.at[0, r, 0],
                wsems[b],
            ).wait()


@jax.jit
def _lookup(xt, weight):
    mesh = plsc.VectorSubcoreMesh(core_axis_name="c", subcore_axis_name="s")
    run = functools.partial(
        pl.kernel,
        mesh=mesh,
        out_type=jax.ShapeDtypeStruct((J, 4, NBLK, BAND), jnp.float32),
        scratch_types=[
            pltpu.VMEM((TPW * CB,), jnp.int32),
            pltpu.VMEM((NBUF, CB, K), jnp.float32),
            pltpu.VMEM((NBUF, K * CB), jnp.float32),
            pltpu.SemaphoreType.DMA,
            pltpu.SemaphoreType.DMA,
            pltpu.SemaphoreType.DMA,
            pltpu.SemaphoreType.DMA,
        ],
        compiler_params=pltpu.CompilerParams(
            use_tc_tiling_on_sc=False, needs_layout_passes=False
        ),
    )(_emb_kernel)
    return run(xt, weight)


def kernel(x, weight):
    xt = jnp.transpose(x).reshape(-1).astype(jnp.int32)
    out5 = _lookup(xt, weight)
    return (out5.reshape(J, 4, NBLK, 4, 8, 128)
                .transpose(2, 3, 5, 0, 1, 4)
                .reshape(NI, J, K))


# speedup vs baseline: 2.8497x; 1.5260x over previous
import functools

import jax
import jax.numpy as jnp
from jax import lax
from jax.experimental import pallas as pl
from jax.experimental.pallas import tpu as pltpu
from jax.experimental.pallas import tpu_sc as plsc

J = 50
NI = 16384
K = 32
CB = 512
NBLK = NI // CB
NTILES = J * NBLK
NC, NS = 2, 16
NW = NC * NS
TPW = NTILES // NW
NBUF = 2


def _emb_kernel(xt_hbm, table_hbm, out5_hbm, idx_all, rows_v, tout_v,
                gsem0, gsem1, wsem0, wsem1):
    wid = lax.axis_index("s") * NC + lax.axis_index("c")
    t0 = wid * TPW
    gsems = (gsem0, gsem1)
    wsems = (wsem0, wsem1)
    pltpu.sync_copy(xt_hbm.at[pl.ds(wid * (TPW * CB), TPW * CB)], idx_all)
    lane = lax.iota(jnp.int32, 16)
    vrow = ((lane >> 3) << 5) + (lane & 7)
    zvec = lane & 0
    for b in range(NBUF):
        pltpu.async_copy(
            table_hbm.at[idx_all.at[pl.ds(b * CB, CB)]], rows_v.at[b], gsems[b]
        )

    @pl.loop(0, TPW, step=NBUF)
    def _body(g):
        for b in range(NBUF):
            n = g + b
            t = t0 + n
            j = t // NBLK
            c = t - j * NBLK
            pltpu.make_async_copy(
                table_hbm.at[idx_all.at[pl.ds(n * CB, CB)]], rows_v.at[b],
                gsems[b],
            ).wait()

            @pl.when(g > 0)
            def _():
                for r in range(4):
                    pltpu.make_async_copy(
                        tout_v.at[b, pl.ds(r * 32, 32), pl.ds(0, 128)],
                        out5_hbm.at[0, r, 0], wsems[b],
                    ).wait()

            for q in range(CB // 128):
                rq_lo = vrow + 8 * q
                rq_hi = rq_lo + 64

                @pl.loop(0, 128, unroll=8, init_carry=zvec)
                def _tr(t, tv):
                    i = q * 128 + t
                    lo = rows_v[b, i, pl.ds(0, 16)]
                    hi = rows_v[b, i, pl.ds(16, 16)]
                    plsc.store_scatter(tout_v.at[b], [rq_lo, tv], lo)
                    plsc.store_scatter(tout_v.at[b], [rq_hi, tv], hi)
                    return tv + 1

            for r in range(4):
                pltpu.async_copy(
                    tout_v.at[b, pl.ds(r * 32, 32), pl.ds(0, 128)],
                    out5_hbm.at[j, r, c], wsems[b],
                )

            @pl.when(n + NBUF < TPW)
            def _():
                pltpu.async_copy(
                    table_hbm.at[idx_all.at[pl.ds((n + NBUF) * CB, CB)]],
                    rows_v.at[b], gsems[b],
                )

    for b in range(NBUF):
        for r in range(4):
            pltpu.make_async_copy(
                tout_v.at[b, pl.ds(r * 32, 32), pl.ds(0, 128)],
                out5_hbm.at[0, r, 0], wsems[b],
            ).wait()


@jax.jit
def _lookup(xt, weight):
    mesh = plsc.VectorSubcoreMesh(core_axis_name="c", subcore_axis_name="s")
    run = functools.partial(
        pl.kernel,
        mesh=mesh,
        out_type=jax.ShapeDtypeStruct((J, 4, NBLK, 32, 128), jnp.float32),
        scratch_types=[
            pltpu.VMEM((TPW * CB,), jnp.int32),
            pltpu.VMEM((NBUF, CB, K), jnp.float32),
            pltpu.VMEM((NBUF, 128, 129), jnp.float32),
            pltpu.SemaphoreType.DMA,
            pltpu.SemaphoreType.DMA,
            pltpu.SemaphoreType.DMA,
            pltpu.SemaphoreType.DMA,
        ],
        compiler_params=pltpu.CompilerParams(
            use_tc_tiling_on_sc=False, needs_layout_passes=False
        ),
    )(_emb_kernel)
    return run(xt, weight)


def kernel(x, weight):
    xt = jnp.transpose(x).reshape(-1).astype(jnp.int32)
    out5 = _lookup(xt, weight)
    return (out5.reshape(J, 4, NBLK, 4, 8, 128)
                .transpose(2, 3, 5, 0, 1, 4)
                .reshape(NI, J, K))


# speedup vs baseline: 2.8518x; 1.0007x over previous
import functools

import jax
import jax.numpy as jnp
from jax import lax
from jax.experimental import pallas as pl
from jax.experimental.pallas import tpu as pltpu
from jax.experimental.pallas import tpu_sc as plsc

J = 50
NI = 16384
K = 32
CB = 512
NBLK = NI // CB
NTILES = J * NBLK
NC, NS = 2, 16
NW = NC * NS
TPW = NTILES // NW
NBUF = 2


def _emb_kernel(xt_hbm, table_hbm, out5_hbm, idx_all, rows_v, tout_v,
                gsem0, gsem1, wsem0, wsem1):
    wid = lax.axis_index("s") * NC + lax.axis_index("c")
    gsems = (gsem0, gsem1)
    wsems = (wsem0, wsem1)
    pltpu.sync_copy(xt_hbm.at[:, pl.ds(wid * CB, CB)], idx_all)
    lane = lax.iota(jnp.int32, 16)
    vrow = ((lane >> 3) << 5) + (lane & 7)
    zvec = lane & 0
    for b in range(NBUF):
        pltpu.async_copy(
            table_hbm.at[idx_all.at[b]], rows_v.at[b], gsems[b]
        )

    @pl.loop(0, TPW, step=NBUF)
    def _body(g):
        for b in range(NBUF):
            j = g + b
            pltpu.make_async_copy(
                table_hbm.at[idx_all.at[j]], rows_v.at[b], gsems[b],
            ).wait()

            @pl.when(g > 0)
            def _():
                for r in range(4):
                    pltpu.make_async_copy(
                        tout_v.at[b, pl.ds(r * 32, 32), pl.ds(0, 128)],
                        out5_hbm.at[0, r, 0], wsems[b],
                    ).wait()

            for q in range(CB // 128):
                rq_lo = vrow + 8 * q
                rq_hi = rq_lo + 64

                @pl.loop(0, 128, unroll=8, init_carry=zvec)
                def _tr(t, tv):
                    i = q * 128 + t
                    lo = rows_v[b, i, pl.ds(0, 16)]
                    hi = rows_v[b, i, pl.ds(16, 16)]
                    plsc.store_scatter(tout_v.at[b], [rq_lo, tv], lo)
                    plsc.store_scatter(tout_v.at[b], [rq_hi, tv], hi)
                    return tv + 1

            for r in range(4):
                pltpu.async_copy(
                    tout_v.at[b, pl.ds(r * 32, 32), pl.ds(0, 128)],
                    out5_hbm.at[j, r, wid], wsems[b],
                )

            @pl.when(j + NBUF < TPW)
            def _():
                pltpu.async_copy(
                    table_hbm.at[idx_all.at[j + NBUF]],
                    rows_v.at[b], gsems[b],
                )

    for b in range(NBUF):
        for r in range(4):
            pltpu.make_async_copy(
                tout_v.at[b, pl.ds(r * 32, 32), pl.ds(0, 128)],
                out5_hbm.at[0, r, 0], wsems[b],
            ).wait()


@jax.jit
def _lookup(xt, weight):
    mesh = plsc.VectorSubcoreMesh(core_axis_name="c", subcore_axis_name="s")
    run = functools.partial(
        pl.kernel,
        mesh=mesh,
        out_type=jax.ShapeDtypeStruct((J, 4, NBLK, 32, 128), jnp.float32),
        scratch_types=[
            pltpu.VMEM((J, CB), jnp.int32),
            pltpu.VMEM((NBUF, CB, K), jnp.float32),
            pltpu.VMEM((NBUF, 128, 129), jnp.float32),
            pltpu.SemaphoreType.DMA,
            pltpu.SemaphoreType.DMA,
            pltpu.SemaphoreType.DMA,
            pltpu.SemaphoreType.DMA,
        ],
        compiler_params=pltpu.CompilerParams(
            use_tc_tiling_on_sc=False, needs_layout_passes=False
        ),
    )(_emb_kernel)
    return run(xt, weight)


def kernel(x, weight):
    xt = jnp.transpose(x).astype(jnp.int32)
    out5 = _lookup(xt, weight)
    return (out5.reshape(J, 4, NBLK, 4, 8, 128)
                .transpose(2, 3, 5, 0, 1, 4)
                .reshape(NI, J, K))
